# Initial kernel scaffold; baseline (speedup 1.0000x reference)
import jax
import jax.numpy as jnp
from jax.experimental import pallas as pl


def kernel(output, nms_thresh):
    raise NotImplementedError("write your pallas kernel here")


# speedup vs baseline: 1.6070x; 1.6070x over previous
import jax
import jax.numpy as jnp
from jax import lax
from jax.experimental import pallas as pl
from jax.experimental.pallas import tpu as pltpu

_NC = 80
_A = 3
_H = 64
_W = 64
_HW = _H * _W
_CH = 5 + _NC
_AW = (12.0 / 32.0, 19.0 / 32.0, 40.0 / 32.0)
_AH = (16.0 / 32.0, 36.0 / 32.0, 28.0 / 32.0)


def _yolo_block(thr_ref, x_ref, boxes_ref, ids_ref):
    ba = pl.program_id(0)
    a = ba % _A
    thr = thr_ref[0, 0]
    x = x_ref[0]

    head = x[0:8, :]
    sg = 1.0 / (1.0 + jnp.exp(-head))
    ex = jnp.exp(head)

    lane = lax.broadcasted_iota(jnp.int32, (1, _HW), 1)
    gx = (lane % _W).astype(jnp.float32)
    gy = (lane // _W).astype(jnp.float32)

    inv_w = jnp.float32(1.0 / _W)
    inv_h = jnp.float32(1.0 / _H)
    aw = jnp.where(
        a == 0, jnp.float32(_AW[0] / _W),
        jnp.where(a == 1, jnp.float32(_AW[1] / _W), jnp.float32(_AW[2] / _W)))
    ah = jnp.where(
        a == 0, jnp.float32(_AH[0] / _H),
        jnp.where(a == 1, jnp.float32(_AH[1] / _H), jnp.float32(_AH[2] / _H)))

    xs = (sg[0:1, :] + gx) * inv_w
    ys = (sg[1:2, :] + gy) * inv_h
    ws = ex[2:3, :] * aw
    hs = ex[3:4, :] * ah
    det = sg[4:5, :]

    sidx = lax.broadcasted_iota(jnp.int32, (_CH, _HW), 0)
    is_cls = sidx >= 5
    xm = jnp.where(is_cls, x, -jnp.inf)
    m = jnp.max(xm, axis=0, keepdims=True)
    p = jnp.exp(x - m)
    s = jnp.sum(jnp.where(is_cls, p, 0.0), axis=0, keepdims=True)
    conf = 1.0 / s
    am = jnp.min(jnp.where(jnp.logical_and(is_cls, xm >= m), sidx, _CH),
                 axis=0, keepdims=True) - 5

    mask = det > thr
    ids_ref[0] = jnp.where(mask, am, -1)

    b8 = jnp.concatenate([xs, ys, ws, hs, det, conf, det, det], axis=0)
    b8 = jnp.where(mask, b8, 0.0)
    boxes_ref[...] = b8.T[:, 0:6]


def kernel(output, nms_thresh):
    b = output.shape[0]
    nba = b * _A
    x3 = output.reshape(nba, _CH, _HW)
    thr = jnp.asarray(nms_thresh, jnp.float32).reshape(1, 1)
    boxes, ids = pl.pallas_call(
        _yolo_block,
        grid=(nba,),
        in_specs=[
            pl.BlockSpec(memory_space=pltpu.SMEM),
            pl.BlockSpec((1, _CH, _HW), lambda i: (i, 0, 0)),
        ],
        out_specs=[
            pl.BlockSpec((_HW, 6), lambda i: (i, 0)),
            pl.BlockSpec((1, 1, _HW), lambda i: (i, 0, 0)),
        ],
        out_shape=[
            jax.ShapeDtypeStruct((nba * _HW, 6), jnp.float32),
            jax.ShapeDtypeStruct((nba, 1, _HW), jnp.int32),
        ],
        compiler_params=pltpu.CompilerParams(
            dimension_semantics=("arbitrary",)),
    )(thr, x3)
    return boxes, ids.reshape(nba * _HW)
